# baseline (speedup 1.0000x reference)
import jax
import jax.numpy as jnp
from jax import lax
from jax.experimental import pallas as pl
from jax.experimental.pallas import tpu as pltpu
from jax.experimental.pallas import tpu_sc as plsc

N_NODES = 10000
N_EDGES = 320000
D = 128
DE = 16
H = 128
NORM = 32.0

NC = 2
NS = 16
NW = NC * NS
CH = 80
NSEG = 5
SEG_E = N_EDGES // NSEG
G_EPW = SEG_E // NW
G_NCH = G_EPW // CH
K = 5
NG = G_NCH // K
S_EPT = SEG_E // NS
S_NCH = S_EPT // CH
N_PAD = 10240
NPT = N_PAD // NS
LANES = 16

_mesh2 = plsc.VectorSubcoreMesh(
    core_axis_name="c", subcore_axis_name="s", num_cores=NC, num_subcores=NS)
_mesh1 = plsc.VectorSubcoreMesh(
    core_axis_name="c", subcore_axis_name="s", num_cores=1, num_subcores=NS)


def _gather_body(a_hbm, b_hbm, rowi_hbm, coli_hbm, out_hbm,
                 idxr_v, idxc_v, b0, b1, b2, b3, b4, s0, s1, s2, s3, s4):
    bufs = (b0, b1, b2, b3, b4)
    sems = (s0, s1, s2, s3, s4)
    cid = lax.axis_index("c")
    sid = lax.axis_index("s")
    base = (cid * NS + sid) * G_EPW
    pltpu.sync_copy(rowi_hbm.at[cid, sid], idxr_v)
    pltpu.sync_copy(coli_hbm.at[cid, sid], idxc_v)

    def group(g, carry):
        js = [g * K + b for b in range(K)]
        for b in range(K):
            @pl.when(g > 0)
            def _(b=b):
                pltpu.make_async_copy(
                    bufs[b], out_hbm.at[pl.ds(base, CH)], sems[b]).wait()
            pltpu.async_copy(a_hbm.at[idxr_v.at[js[b]]], bufs[b], sems[b])
        for b in range(K):
            pltpu.make_async_copy(
                a_hbm.at[idxr_v.at[js[b]]], bufs[b], sems[b]).wait()
            pltpu.async_copy(b_hbm.at[idxc_v.at[js[b]]], bufs[b], sems[b],
                             add=True)
        for b in range(K):
            pltpu.make_async_copy(
                b_hbm.at[idxc_v.at[js[b]]], bufs[b], sems[b]).wait()
            pltpu.async_copy(bufs[b],
                             out_hbm.at[pl.ds(base + js[b] * CH, CH)], sems[b])
        return carry

    lax.fori_loop(0, NG, group, 0)
    for b in range(K):
        pltpu.make_async_copy(bufs[b], out_hbm.at[pl.ds(base, CH)],
                              sems[b]).wait()


_gather = pl.kernel(
    _gather_body,
    out_type=jax.ShapeDtypeStruct((SEG_E, D), jnp.float32),
    mesh=_mesh2,
    scratch_types=[
        pltpu.VMEM((G_NCH, CH), jnp.int32),
        pltpu.VMEM((G_NCH, CH), jnp.int32),
    ] + [pltpu.VMEM((CH, D), jnp.float32)] * K
      + [pltpu.SemaphoreType.DMA] * K,
)


HH = H // 2


def _scatter_half(ms, out_hbm, sid, rowi_hbm, idxr_v, mbuf_v, mbuf2_v, acc_s,
                  sem):
    def zrow(r, carry):
        for c8 in range(HH // LANES):
            mbuf_v[r, pl.ds(c8 * LANES, LANES)] = jnp.zeros((LANES,), jnp.float32)
        return carry

    lax.fori_loop(0, CH, zrow, 0)

    def zchunk(k, carry):
        pltpu.sync_copy(mbuf_v, acc_s.at[pl.ds(sid * NPT + k * CH, CH)])
        return carry

    lax.fori_loop(0, NPT // CH, zchunk, 0)
    plsc.subcore_barrier()

    for seg, m_hbm in enumerate(ms):
        base = sid * S_EPT
        pltpu.sync_copy(rowi_hbm.at[seg, sid], idxr_v)
        pltpu.async_copy(m_hbm.at[pl.ds(base, CH)], mbuf_v, sem)

        def chunk(j, carry, m_hbm=m_hbm, base=base):
            nxt = jnp.minimum(j + 1, S_NCH - 1)
            cur_is_0 = lax.rem(j, 2) == 0

            @pl.when(cur_is_0)
            def _():
                pltpu.make_async_copy(
                    m_hbm.at[pl.ds(base, CH)], mbuf_v, sem).wait()
                pltpu.async_copy(
                    m_hbm.at[pl.ds(base + nxt * CH, CH)], mbuf2_v, sem)
                pltpu.sync_copy(mbuf_v, acc_s.at[idxr_v.at[j]], add=True)

            @pl.when(jnp.logical_not(cur_is_0))
            def _():
                pltpu.make_async_copy(
                    m_hbm.at[pl.ds(base, CH)], mbuf2_v, sem).wait()
                pltpu.async_copy(
                    m_hbm.at[pl.ds(base + nxt * CH, CH)], mbuf_v, sem)
                pltpu.sync_copy(mbuf2_v, acc_s.at[idxr_v.at[j]], add=True)

            return carry

        lax.fori_loop(0, S_NCH, chunk, 0)
        pltpu.make_async_copy(
            m_hbm.at[pl.ds(base, CH)],
            mbuf_v if S_NCH % 2 == 0 else mbuf2_v, sem).wait()

    plsc.subcore_barrier()

    def dchunk(k, carry):
        pltpu.sync_copy(acc_s.at[pl.ds(sid * NPT + k * CH, CH)], mbuf_v)
        pltpu.sync_copy(mbuf_v, out_hbm.at[pl.ds(sid * NPT + k * CH, CH)])
        return carry

    lax.fori_loop(0, NPT // CH, dchunk, 0)


def _scatter_body(ml0, ml1, ml2, ml3, ml4, mh0, mh1, mh2, mh3, mh4,
                  rowi_hbm, out_lo_hbm, out_hi_hbm,
                  idxr_v, mbuf_v, mbuf2_v, acc_s, sem):
    cid = lax.axis_index("c")
    sid = lax.axis_index("s")

    @pl.when(cid == 0)
    def _():
        _scatter_half((ml0, ml1, ml2, ml3, ml4), out_lo_hbm, sid, rowi_hbm,
                      idxr_v, mbuf_v, mbuf2_v, acc_s, sem)

    @pl.when(cid == 1)
    def _():
        _scatter_half((mh0, mh1, mh2, mh3, mh4), out_hi_hbm, sid, rowi_hbm,
                      idxr_v, mbuf_v, mbuf2_v, acc_s, sem)


_scatter = pl.kernel(
    _scatter_body,
    out_type=[jax.ShapeDtypeStruct((N_PAD, HH), jnp.float32),
              jax.ShapeDtypeStruct((N_PAD, HH), jnp.float32)],
    mesh=_mesh2,
    scratch_types=[
        pltpu.VMEM((S_NCH, CH), jnp.int32),
        pltpu.VMEM((CH, HH), jnp.float32),
        pltpu.VMEM((CH, HH), jnp.float32),
        pltpu.MemorySpace.VMEM_SHARED((N_PAD, HH), jnp.float32),
        pltpu.SemaphoreType.DMA,
    ],
)


def _precompute_body(h_ref, w1a_ref, w1b_ref, b1_ref, a_ref, b_ref):
    hh = h_ref[...]
    a_ref[...] = jnp.dot(hh, w1a_ref[...],
                         preferred_element_type=jnp.float32) + b1_ref[...]
    b_ref[...] = jnp.dot(hh, w1b_ref[...], preferred_element_type=jnp.float32)


def _edge_mlp_body(pre_ref, ea_ref, w1c_ref, w2_ref, b2_ref, wat_ref, ba_ref,
                   out_lo_ref, out_hi_ref):
    x = pre_ref[...] + jnp.dot(ea_ref[...], w1c_ref[...],
                               preferred_element_type=jnp.float32)
    m = jnp.maximum(x, 0.0)
    m = jnp.maximum(
        jnp.dot(m, w2_ref[...], preferred_element_type=jnp.float32)
        + b2_ref[...], 0.0)
    logit = jnp.sum(m * wat_ref[...], axis=1, keepdims=True) + ba_ref[...]
    m = m * jax.nn.sigmoid(logit)
    out_lo_ref[...] = m[:, :HH]
    out_hi_ref[...] = m[:, HH:]


def _node_body(h_ref, plo_ref, phi_ref, wu1h_ref, wu1a_lo_ref, wu1a_hi_ref,
               bu1_ref, wu2_ref, bu2_ref, out_ref):
    hh = h_ref[...]
    u = jnp.maximum(
        jnp.dot(hh, wu1h_ref[...], preferred_element_type=jnp.float32)
        + jnp.dot(plo_ref[...] * (1.0 / NORM), wu1a_lo_ref[...],
                  preferred_element_type=jnp.float32)
        + jnp.dot(phi_ref[...] * (1.0 / NORM), wu1a_hi_ref[...],
                  preferred_element_type=jnp.float32)
        + bu1_ref[...], 0.0)
    out_ref[...] = hh + jnp.dot(u, wu2_ref[...],
                                preferred_element_type=jnp.float32) + bu2_ref[...]


BE = 8000
BN = 1000


def kernel(h, edge_index, edge_attr, W1, b1, W2, b2, Wa, ba, Wu1, bu1, Wu2, bu2):
    W1a, W1b, W1c = W1[:D], W1[D:2 * D], W1[2 * D:]
    Wu1h, Wu1a = Wu1[:D], Wu1[D:]
    row = edge_index[0]
    row_g = row.reshape(NSEG, NC, NS, G_NCH, CH)
    col_g = edge_index[1].reshape(NSEG, NC, NS, G_NCH, CH)
    row_s = row.reshape(NSEG, NS, S_NCH, CH)

    A, B = pl.pallas_call(
        _precompute_body,
        out_shape=[jax.ShapeDtypeStruct((N_NODES, D), jnp.float32),
                   jax.ShapeDtypeStruct((N_NODES, D), jnp.float32)],
    )(h, W1a, W1b, b1.reshape(1, H))

    zero = lambda i: (0, 0)
    nb_seg = SEG_E // BE

    def edge_mlp_seg(pre_s, ea_off):
        return pl.pallas_call(
            _edge_mlp_body,
            grid=(nb_seg,),
            in_specs=[
                pl.BlockSpec((BE, D), lambda i: (i, 0)),
                pl.BlockSpec((BE, DE), lambda i: (i + ea_off, 0)),
                pl.BlockSpec((DE, H), zero),
                pl.BlockSpec((H, H), zero),
                pl.BlockSpec((1, H), zero),
                pl.BlockSpec((1, H), zero),
                pl.BlockSpec((1, 1), zero),
            ],
            out_specs=[pl.BlockSpec((BE, HH), lambda i: (i, 0)),
                       pl.BlockSpec((BE, HH), lambda i: (i, 0))],
            out_shape=[jax.ShapeDtypeStruct((SEG_E, HH), jnp.float32),
                       jax.ShapeDtypeStruct((SEG_E, HH), jnp.float32)],
        )(pre_s, edge_attr, W1c, W2, b2.reshape(1, H), Wa.reshape(1, H),
          ba.reshape(1, 1))

    mlos, mhis = [], []
    for s in range(NSEG):
        pre_s = _gather(A, B, row_g[s], col_g[s])
        mlo, mhi = edge_mlp_seg(pre_s, s * nb_seg)
        mlos.append(mlo)
        mhis.append(mhi)

    agg_lo, agg_hi = _scatter(*mlos, *mhis, row_s)
    agg_lo = agg_lo[:N_NODES]
    agg_hi = agg_hi[:N_NODES]

    out = pl.pallas_call(
        _node_body,
        grid=(N_NODES // BN,),
        in_specs=[
            pl.BlockSpec((BN, D), lambda i: (i, 0)),
            pl.BlockSpec((BN, HH), lambda i: (i, 0)),
            pl.BlockSpec((BN, HH), lambda i: (i, 0)),
            pl.BlockSpec((H, H), zero),
            pl.BlockSpec((HH, H), zero),
            pl.BlockSpec((HH, H), zero),
            pl.BlockSpec((1, H), zero),
            pl.BlockSpec((H, H), zero),
            pl.BlockSpec((1, H), zero),
        ],
        out_specs=pl.BlockSpec((BN, D), lambda i: (i, 0)),
        out_shape=jax.ShapeDtypeStruct((N_NODES, D), jnp.float32),
    )(h, agg_lo, agg_hi, Wu1h, Wu1a[:HH], Wu1a[HH:], bu1.reshape(1, H), Wu2,
      bu2.reshape(1, H))

    return out


# speedup vs baseline: 1.0723x; 1.0723x over previous
import jax
import jax.numpy as jnp
from jax import lax
from jax.experimental import pallas as pl
from jax.experimental.pallas import tpu as pltpu
from jax.experimental.pallas import tpu_sc as plsc

N_NODES = 10000
N_EDGES = 320000
D = 128
DE = 16
H = 128
NORM = 32.0

NC = 2
NS = 16
NW = NC * NS
CH = 80
NSEG = 5
SEG_E = N_EDGES // NSEG
G_EPW = SEG_E // NW
G_NCH = G_EPW // CH
K = 5
NG = G_NCH // K
S_EPT = SEG_E // NS
S_NCH = S_EPT // CH
N_PAD = 10240
NPT = N_PAD // NS
LANES = 16

_mesh2 = plsc.VectorSubcoreMesh(
    core_axis_name="c", subcore_axis_name="s", num_cores=NC, num_subcores=NS)
_mesh1 = plsc.VectorSubcoreMesh(
    core_axis_name="c", subcore_axis_name="s", num_cores=1, num_subcores=NS)


def _gather_body(a_hbm, b_hbm, rowi_hbm, coli_hbm, out_hbm,
                 idxr_v, idxc_v, b0, b1, b2, b3, b4, s0, s1, s2, s3, s4):
    bufs = (b0, b1, b2, b3, b4)
    sems = (s0, s1, s2, s3, s4)
    cid = lax.axis_index("c")
    sid = lax.axis_index("s")
    base = (cid * NS + sid) * G_EPW
    pltpu.sync_copy(rowi_hbm.at[cid, sid], idxr_v)
    pltpu.sync_copy(coli_hbm.at[cid, sid], idxc_v)

    def group(g, carry):
        js = [g * K + b for b in range(K)]
        for b in range(K):
            @pl.when(g > 0)
            def _(b=b):
                pltpu.make_async_copy(
                    bufs[b], out_hbm.at[pl.ds(base, CH)], sems[b]).wait()
            pltpu.async_copy(a_hbm.at[idxr_v.at[js[b]]], bufs[b], sems[b])
        for b in range(K):
            pltpu.make_async_copy(
                a_hbm.at[idxr_v.at[js[b]]], bufs[b], sems[b]).wait()
            pltpu.async_copy(b_hbm.at[idxc_v.at[js[b]]], bufs[b], sems[b],
                             add=True)
        for b in range(K):
            pltpu.make_async_copy(
                b_hbm.at[idxc_v.at[js[b]]], bufs[b], sems[b]).wait()
            pltpu.async_copy(bufs[b],
                             out_hbm.at[pl.ds(base + js[b] * CH, CH)], sems[b])
        return carry

    lax.fori_loop(0, NG, group, 0)
    for b in range(K):
        pltpu.make_async_copy(bufs[b], out_hbm.at[pl.ds(base, CH)],
                              sems[b]).wait()


_gather = pl.kernel(
    _gather_body,
    out_type=jax.ShapeDtypeStruct((SEG_E, D), jnp.float32),
    mesh=_mesh2,
    scratch_types=[
        pltpu.VMEM((G_NCH, CH), jnp.int32),
        pltpu.VMEM((G_NCH, CH), jnp.int32),
    ] + [pltpu.VMEM((CH, D), jnp.float32)] * K
      + [pltpu.SemaphoreType.DMA] * K,
)


def _scatter_body(m0_hbm, m1_hbm, m2_hbm, m3_hbm, m4_hbm, rowi_hbm, out_hbm,
                  idxr_v, mbuf_v, mbuf2_v, acc_s, sem):
    sid = lax.axis_index("s")

    def zrow(r, carry):
        for c8 in range(D // LANES):
            mbuf_v[r, pl.ds(c8 * LANES, LANES)] = jnp.zeros((LANES,), jnp.float32)
        return carry

    lax.fori_loop(0, CH, zrow, 0)

    def zchunk(k, carry):
        pltpu.sync_copy(mbuf_v, acc_s.at[pl.ds(sid * NPT + k * CH, CH)])
        return carry

    lax.fori_loop(0, NPT // CH, zchunk, 0)
    plsc.subcore_barrier()

    for seg, m_hbm in enumerate((m0_hbm, m1_hbm, m2_hbm, m3_hbm, m4_hbm)):
        base = sid * S_EPT
        pltpu.sync_copy(rowi_hbm.at[seg, sid], idxr_v)
        pltpu.async_copy(m_hbm.at[pl.ds(base, CH)], mbuf_v, sem)

        def chunk(j, carry, m_hbm=m_hbm, base=base):
            nxt = jnp.minimum(j + 1, S_NCH - 1)
            cur_is_0 = lax.rem(j, 2) == 0

            @pl.when(cur_is_0)
            def _():
                pltpu.make_async_copy(
                    m_hbm.at[pl.ds(base, CH)], mbuf_v, sem).wait()
                pltpu.async_copy(
                    m_hbm.at[pl.ds(base + nxt * CH, CH)], mbuf2_v, sem)
                pltpu.sync_copy(mbuf_v, acc_s.at[idxr_v.at[j]], add=True)

            @pl.when(jnp.logical_not(cur_is_0))
            def _():
                pltpu.make_async_copy(
                    m_hbm.at[pl.ds(base, CH)], mbuf2_v, sem).wait()
                pltpu.async_copy(
                    m_hbm.at[pl.ds(base + nxt * CH, CH)], mbuf_v, sem)
                pltpu.sync_copy(mbuf2_v, acc_s.at[idxr_v.at[j]], add=True)

            return carry

        lax.fori_loop(0, S_NCH, chunk, 0)
        pltpu.make_async_copy(
            m_hbm.at[pl.ds(base, CH)],
            mbuf_v if S_NCH % 2 == 0 else mbuf2_v, sem).wait()

    plsc.subcore_barrier()

    def dchunk(k, carry):
        pltpu.sync_copy(acc_s.at[pl.ds(sid * NPT + k * CH, CH)], mbuf_v)
        pltpu.sync_copy(mbuf_v, out_hbm.at[pl.ds(sid * NPT + k * CH, CH)])
        return carry

    lax.fori_loop(0, NPT // CH, dchunk, 0)


_scatter = pl.kernel(
    _scatter_body,
    out_type=jax.ShapeDtypeStruct((N_PAD, D), jnp.float32),
    mesh=_mesh1,
    scratch_types=[
        pltpu.VMEM((S_NCH, CH), jnp.int32),
        pltpu.VMEM((CH, D), jnp.float32),
        pltpu.VMEM((CH, D), jnp.float32),
        pltpu.MemorySpace.VMEM_SHARED((N_PAD, D), jnp.float32),
        pltpu.SemaphoreType.DMA,
    ],
)


def _precompute_body(h_ref, w1a_ref, w1b_ref, b1_ref, a_ref, b_ref):
    hh = h_ref[...]
    a_ref[...] = jnp.dot(hh, w1a_ref[...],
                         preferred_element_type=jnp.float32) + b1_ref[...]
    b_ref[...] = jnp.dot(hh, w1b_ref[...], preferred_element_type=jnp.float32)


def _edge_mlp_body(pre_ref, ea_ref, w1c_ref, w2_ref, b2_ref, wat_ref, ba_ref,
                   out_ref):
    x = pre_ref[...] + jnp.dot(ea_ref[...], w1c_ref[...],
                               preferred_element_type=jnp.float32)
    m = jnp.maximum(x, 0.0)
    m = jnp.maximum(
        jnp.dot(m, w2_ref[...], preferred_element_type=jnp.float32)
        + b2_ref[...], 0.0)
    logit = jnp.sum(m * wat_ref[...], axis=1, keepdims=True) + ba_ref[...]
    out_ref[...] = m * jax.nn.sigmoid(logit)


def _node_body(h_ref, p_ref, wu1h_ref, wu1a_ref, bu1_ref, wu2_ref, bu2_ref,
               out_ref):
    hh = h_ref[...]
    u = jnp.maximum(
        jnp.dot(hh, wu1h_ref[...], preferred_element_type=jnp.float32)
        + jnp.dot(p_ref[...] * (1.0 / NORM), wu1a_ref[...],
                  preferred_element_type=jnp.float32)
        + bu1_ref[...], 0.0)
    out_ref[...] = hh + jnp.dot(u, wu2_ref[...],
                                preferred_element_type=jnp.float32) + bu2_ref[...]


BE = 8000
BN = 1000


def kernel(h, edge_index, edge_attr, W1, b1, W2, b2, Wa, ba, Wu1, bu1, Wu2, bu2):
    W1a, W1b, W1c = W1[:D], W1[D:2 * D], W1[2 * D:]
    Wu1h, Wu1a = Wu1[:D], Wu1[D:]
    row = edge_index[0]
    row_g = row.reshape(NSEG, NC, NS, G_NCH, CH)
    col_g = edge_index[1].reshape(NSEG, NC, NS, G_NCH, CH)
    row_s = row.reshape(NSEG, NS, S_NCH, CH)

    A, B = pl.pallas_call(
        _precompute_body,
        out_shape=[jax.ShapeDtypeStruct((N_NODES, D), jnp.float32),
                   jax.ShapeDtypeStruct((N_NODES, D), jnp.float32)],
    )(h, W1a, W1b, b1.reshape(1, H))

    zero = lambda i: (0, 0)
    nb_seg = SEG_E // BE

    def edge_mlp_seg(pre_s, ea_off):
        return pl.pallas_call(
            _edge_mlp_body,
            grid=(nb_seg,),
            in_specs=[
                pl.BlockSpec((BE, D), lambda i: (i, 0)),
                pl.BlockSpec((BE, DE), lambda i: (i + ea_off, 0)),
                pl.BlockSpec((DE, H), zero),
                pl.BlockSpec((H, H), zero),
                pl.BlockSpec((1, H), zero),
                pl.BlockSpec((1, H), zero),
                pl.BlockSpec((1, 1), zero),
            ],
            out_specs=pl.BlockSpec((BE, D), lambda i: (i, 0)),
            out_shape=jax.ShapeDtypeStruct((SEG_E, D), jnp.float32),
        )(pre_s, edge_attr, W1c, W2, b2.reshape(1, H), Wa.reshape(1, H),
          ba.reshape(1, 1))

    mms = []
    for s in range(NSEG):
        pre_s = _gather(A, B, row_g[s], col_g[s])
        mms.append(edge_mlp_seg(pre_s, s * nb_seg))

    agg = _scatter(*mms, row_s)[:N_NODES]

    out = pl.pallas_call(
        _node_body,
        grid=(N_NODES // BN,),
        in_specs=[
            pl.BlockSpec((BN, D), lambda i: (i, 0)),
            pl.BlockSpec((BN, D), lambda i: (i, 0)),
            pl.BlockSpec((H, H), zero),
            pl.BlockSpec((H, H), zero),
            pl.BlockSpec((1, H), zero),
            pl.BlockSpec((H, H), zero),
            pl.BlockSpec((1, H), zero),
        ],
        out_specs=pl.BlockSpec((BN, D), lambda i: (i, 0)),
        out_shape=jax.ShapeDtypeStruct((N_NODES, D), jnp.float32),
    )(h, agg, Wu1h, Wu1a, bu1.reshape(1, H), Wu2, bu2.reshape(1, H))

    return out


# speedup vs baseline: 1.0742x; 1.0018x over previous
import jax
import jax.numpy as jnp
from jax import lax
from jax.experimental import pallas as pl
from jax.experimental.pallas import tpu as pltpu
from jax.experimental.pallas import tpu_sc as plsc

N_NODES = 10000
N_EDGES = 320000
D = 128
DE = 16
H = 128
NORM = 32.0

NC = 2
NS = 16
NW = NC * NS
CH = 80
NSEG = 5
SEG_E = N_EDGES // NSEG
G_EPW = SEG_E // NW
G_NCH = G_EPW // CH
K = 5
NG = G_NCH // K
S_EPT = SEG_E // NS
S_CH = 80
S_NCH = S_EPT // S_CH
N_PAD = 10240
NPT = N_PAD // NS
NPT_F = NPT // S_CH
NPT_T = 0
LANES = 16

_mesh2 = plsc.VectorSubcoreMesh(
    core_axis_name="c", subcore_axis_name="s", num_cores=NC, num_subcores=NS)
_mesh1 = plsc.VectorSubcoreMesh(
    core_axis_name="c", subcore_axis_name="s", num_cores=1, num_subcores=NS)


def _gather_body(a_hbm, b_hbm, rowi_hbm, coli_hbm, out_hbm,
                 idxr_v, idxc_v, b0, b1, b2, b3, b4, s0, s1, s2, s3, s4):
    bufs = (b0, b1, b2, b3, b4)
    sems = (s0, s1, s2, s3, s4)
    cid = lax.axis_index("c")
    sid = lax.axis_index("s")
    base = (cid * NS + sid) * G_EPW
    pltpu.sync_copy(rowi_hbm.at[cid, sid], idxr_v)
    pltpu.sync_copy(coli_hbm.at[cid, sid], idxc_v)

    def group(g, carry):
        js = [g * K + b for b in range(K)]
        for b in range(K):
            @pl.when(g > 0)
            def _(b=b):
                pltpu.make_async_copy(
                    bufs[b], out_hbm.at[pl.ds(base, CH)], sems[b]).wait()
            pltpu.async_copy(a_hbm.at[idxr_v.at[js[b]]], bufs[b], sems[b])
        for b in range(K):
            pltpu.make_async_copy(
                a_hbm.at[idxr_v.at[js[b]]], bufs[b], sems[b]).wait()
            pltpu.async_copy(b_hbm.at[idxc_v.at[js[b]]], bufs[b], sems[b],
                             add=True)
        for b in range(K):
            pltpu.make_async_copy(
                b_hbm.at[idxc_v.at[js[b]]], bufs[b], sems[b]).wait()
            pltpu.async_copy(bufs[b],
                             out_hbm.at[pl.ds(base + js[b] * CH, CH)], sems[b])
        return carry

    lax.fori_loop(0, NG, group, 0)
    for b in range(K):
        pltpu.make_async_copy(bufs[b], out_hbm.at[pl.ds(base, CH)],
                              sems[b]).wait()


_gather = pl.kernel(
    _gather_body,
    out_type=jax.ShapeDtypeStruct((SEG_E, D), jnp.float32),
    mesh=_mesh2,
    scratch_types=[
        pltpu.VMEM((G_NCH, CH), jnp.int32),
        pltpu.VMEM((G_NCH, CH), jnp.int32),
    ] + [pltpu.VMEM((CH, D), jnp.float32)] * K
      + [pltpu.SemaphoreType.DMA] * K,
)


def _scatter_body(m0_hbm, m1_hbm, m2_hbm, m3_hbm, m4_hbm, rowi_hbm, out_hbm,
                  idxr_v, mbuf_v, mbuf2_v, acc_s, sem):
    sid = lax.axis_index("s")

    def zrow(r, carry):
        for c8 in range(D // LANES):
            mbuf_v[r, pl.ds(c8 * LANES, LANES)] = jnp.zeros((LANES,), jnp.float32)
        return carry

    lax.fori_loop(0, S_CH, zrow, 0)

    def zchunk(k, carry):
        pltpu.sync_copy(mbuf_v, acc_s.at[pl.ds(sid * NPT + k * S_CH, S_CH)])
        return carry

    lax.fori_loop(0, NPT_F, zchunk, 0)
    if NPT_T:
        pltpu.sync_copy(mbuf_v.at[pl.ds(0, NPT_T)],
                        acc_s.at[pl.ds(sid * NPT + NPT_F * S_CH, NPT_T)])
    plsc.subcore_barrier()

    for seg, m_hbm in enumerate((m0_hbm, m1_hbm, m2_hbm, m3_hbm, m4_hbm)):
        base = sid * S_EPT
        pltpu.sync_copy(rowi_hbm.at[seg, sid], idxr_v)
        pltpu.async_copy(m_hbm.at[pl.ds(base, S_CH)], mbuf_v, sem)

        def chunk(j, carry, m_hbm=m_hbm, base=base):
            nxt = jnp.minimum(j + 1, S_NCH - 1)
            cur_is_0 = lax.rem(j, 2) == 0

            @pl.when(cur_is_0)
            def _():
                pltpu.make_async_copy(
                    m_hbm.at[pl.ds(base, S_CH)], mbuf_v, sem).wait()
                pltpu.async_copy(
                    m_hbm.at[pl.ds(base + nxt * S_CH, S_CH)], mbuf2_v, sem)
                pltpu.sync_copy(mbuf_v, acc_s.at[idxr_v.at[j]], add=True)

            @pl.when(jnp.logical_not(cur_is_0))
            def _():
                pltpu.make_async_copy(
                    m_hbm.at[pl.ds(base, S_CH)], mbuf2_v, sem).wait()
                pltpu.async_copy(
                    m_hbm.at[pl.ds(base + nxt * S_CH, S_CH)], mbuf_v, sem)
                pltpu.sync_copy(mbuf2_v, acc_s.at[idxr_v.at[j]], add=True)

            return carry

        lax.fori_loop(0, S_NCH, chunk, 0)
        pltpu.make_async_copy(
            m_hbm.at[pl.ds(base, S_CH)],
            mbuf_v if S_NCH % 2 == 0 else mbuf2_v, sem).wait()

    plsc.subcore_barrier()

    def dchunk(k, carry):
        pltpu.sync_copy(acc_s.at[pl.ds(sid * NPT + k * S_CH, S_CH)], mbuf_v)
        pltpu.sync_copy(mbuf_v, out_hbm.at[pl.ds(sid * NPT + k * S_CH, S_CH)])
        return carry

    lax.fori_loop(0, NPT_F, dchunk, 0)
    if NPT_T:
        pltpu.sync_copy(acc_s.at[pl.ds(sid * NPT + NPT_F * S_CH, NPT_T)],
                        mbuf_v.at[pl.ds(0, NPT_T)])
        pltpu.sync_copy(mbuf_v.at[pl.ds(0, NPT_T)],
                        out_hbm.at[pl.ds(sid * NPT + NPT_F * S_CH, NPT_T)])


_scatter = pl.kernel(
    _scatter_body,
    out_type=jax.ShapeDtypeStruct((N_PAD, D), jnp.float32),
    mesh=_mesh1,
    scratch_types=[
        pltpu.VMEM((S_NCH, S_CH), jnp.int32),
        pltpu.VMEM((S_CH, D), jnp.float32),
        pltpu.VMEM((S_CH, D), jnp.float32),
        pltpu.MemorySpace.VMEM_SHARED((N_PAD, D), jnp.float32),
        pltpu.SemaphoreType.DMA,
    ],
)


def _precompute_body(h_ref, w1a_ref, w1b_ref, b1_ref, a_ref, b_ref):
    hh = h_ref[...]
    a_ref[...] = jnp.dot(hh, w1a_ref[...],
                         preferred_element_type=jnp.float32) + b1_ref[...]
    b_ref[...] = jnp.dot(hh, w1b_ref[...], preferred_element_type=jnp.float32)


def _edge_mlp_body(pre_ref, ea_ref, w1c_ref, w2_ref, b2_ref, wat_ref, ba_ref,
                   out_ref):
    x = pre_ref[...] + jnp.dot(ea_ref[...], w1c_ref[...],
                               preferred_element_type=jnp.float32)
    m = jnp.maximum(x, 0.0)
    m = jnp.maximum(
        jnp.dot(m, w2_ref[...], preferred_element_type=jnp.float32)
        + b2_ref[...], 0.0)
    logit = jnp.sum(m * wat_ref[...], axis=1, keepdims=True) + ba_ref[...]
    out_ref[...] = m * jax.nn.sigmoid(logit)


def _node_body(h_ref, p_ref, wu1h_ref, wu1a_ref, bu1_ref, wu2_ref, bu2_ref,
               out_ref):
    hh = h_ref[...]
    u = jnp.maximum(
        jnp.dot(hh, wu1h_ref[...], preferred_element_type=jnp.float32)
        + jnp.dot(p_ref[...] * (1.0 / NORM), wu1a_ref[...],
                  preferred_element_type=jnp.float32)
        + bu1_ref[...], 0.0)
    out_ref[...] = hh + jnp.dot(u, wu2_ref[...],
                                preferred_element_type=jnp.float32) + bu2_ref[...]


BE = 8000
BN = 1000


def kernel(h, edge_index, edge_attr, W1, b1, W2, b2, Wa, ba, Wu1, bu1, Wu2, bu2):
    W1a, W1b, W1c = W1[:D], W1[D:2 * D], W1[2 * D:]
    Wu1h, Wu1a = Wu1[:D], Wu1[D:]
    row = edge_index[0]
    row_g = row.reshape(NSEG, NC, NS, G_NCH, CH)
    col_g = edge_index[1].reshape(NSEG, NC, NS, G_NCH, CH)
    row_s = row.reshape(NSEG, NS, S_NCH, S_CH)

    A, B = pl.pallas_call(
        _precompute_body,
        out_shape=[jax.ShapeDtypeStruct((N_NODES, D), jnp.float32),
                   jax.ShapeDtypeStruct((N_NODES, D), jnp.float32)],
    )(h, W1a, W1b, b1.reshape(1, H))

    zero = lambda i: (0, 0)
    nb_seg = SEG_E // BE

    def edge_mlp_seg(pre_s, ea_off):
        return pl.pallas_call(
            _edge_mlp_body,
            grid=(nb_seg,),
            in_specs=[
                pl.BlockSpec((BE, D), lambda i: (i, 0)),
                pl.BlockSpec((BE, DE), lambda i: (i + ea_off, 0)),
                pl.BlockSpec((DE, H), zero),
                pl.BlockSpec((H, H), zero),
                pl.BlockSpec((1, H), zero),
                pl.BlockSpec((1, H), zero),
                pl.BlockSpec((1, 1), zero),
            ],
            out_specs=pl.BlockSpec((BE, D), lambda i: (i, 0)),
            out_shape=jax.ShapeDtypeStruct((SEG_E, D), jnp.float32),
        )(pre_s, edge_attr, W1c, W2, b2.reshape(1, H), Wa.reshape(1, H),
          ba.reshape(1, 1))

    mms = []
    for s in range(NSEG):
        pre_s = _gather(A, B, row_g[s], col_g[s])
        mms.append(edge_mlp_seg(pre_s, s * nb_seg))

    agg = _scatter(*mms, row_s)[:N_NODES]

    out = pl.pallas_call(
        _node_body,
        grid=(N_NODES // BN,),
        in_specs=[
            pl.BlockSpec((BN, D), lambda i: (i, 0)),
            pl.BlockSpec((BN, D), lambda i: (i, 0)),
            pl.BlockSpec((H, H), zero),
            pl.BlockSpec((H, H), zero),
            pl.BlockSpec((1, H), zero),
            pl.BlockSpec((H, H), zero),
            pl.BlockSpec((1, H), zero),
        ],
        out_specs=pl.BlockSpec((BN, D), lambda i: (i, 0)),
        out_shape=jax.ShapeDtypeStruct((N_NODES, D), jnp.float32),
    )(h, agg, Wu1h, Wu1a, bu1.reshape(1, H), Wu2, bu2.reshape(1, H))

    return out


# speedup vs baseline: 1.0891x; 1.0139x over previous
import jax
import jax.numpy as jnp
from jax import lax
from jax.experimental import pallas as pl
from jax.experimental.pallas import tpu as pltpu
from jax.experimental.pallas import tpu_sc as plsc

N_NODES = 10000
N_EDGES = 320000
D = 128
DE = 16
H = 128
NORM = 32.0

NC = 2
NS = 16
NW = NC * NS
CH = 80
SEG_ES = (64000, 128000, 128000)
NSEG = len(SEG_ES)
K = 5
S_CH = 80
S_NCH_MAX = max(s // NS // S_CH for s in SEG_ES)
N_PAD = 10240
NPT = N_PAD // NS
NPT_F = NPT // S_CH
NPT_T = 0
LANES = 16

_mesh2 = plsc.VectorSubcoreMesh(
    core_axis_name="c", subcore_axis_name="s", num_cores=NC, num_subcores=NS)
_mesh1 = plsc.VectorSubcoreMesh(
    core_axis_name="c", subcore_axis_name="s", num_cores=1, num_subcores=NS)


def _make_gather(seg_e):
    g_epw = seg_e // NW
    g_nch = g_epw // CH
    ng = g_nch // K

    def _gather_body(a_hbm, b_hbm, rowi_hbm, coli_hbm, out_hbm,
                     idxr_v, idxc_v, b0, b1, b2, b3, b4, s0, s1, s2, s3, s4):
        bufs = (b0, b1, b2, b3, b4)
        sems = (s0, s1, s2, s3, s4)
        cid = lax.axis_index("c")
        sid = lax.axis_index("s")
        base = (cid * NS + sid) * g_epw
        pltpu.sync_copy(rowi_hbm.at[cid, sid], idxr_v)
        pltpu.sync_copy(coli_hbm.at[cid, sid], idxc_v)

        def group(g, carry):
            js = [g * K + b for b in range(K)]
            for b in range(K):
                @pl.when(g > 0)
                def _(b=b):
                    pltpu.make_async_copy(
                        bufs[b], out_hbm.at[pl.ds(base, CH)], sems[b]).wait()
                pltpu.async_copy(a_hbm.at[idxr_v.at[js[b]]], bufs[b], sems[b])
            for b in range(K):
                pltpu.make_async_copy(
                    a_hbm.at[idxr_v.at[js[b]]], bufs[b], sems[b]).wait()
                pltpu.async_copy(b_hbm.at[idxc_v.at[js[b]]], bufs[b], sems[b],
                                 add=True)
            for b in range(K):
                pltpu.make_async_copy(
                    b_hbm.at[idxc_v.at[js[b]]], bufs[b], sems[b]).wait()
                pltpu.async_copy(
                    bufs[b], out_hbm.at[pl.ds(base + js[b] * CH, CH)], sems[b])
            return carry

        lax.fori_loop(0, ng, group, 0)
        for b in range(K):
            pltpu.make_async_copy(bufs[b], out_hbm.at[pl.ds(base, CH)],
                                  sems[b]).wait()

    return pl.kernel(
        _gather_body,
        out_type=jax.ShapeDtypeStruct((seg_e, D), jnp.float32),
        mesh=_mesh2,
        scratch_types=[
            pltpu.VMEM((g_nch, CH), jnp.int32),
            pltpu.VMEM((g_nch, CH), jnp.int32),
        ] + [pltpu.VMEM((CH, D), jnp.float32)] * K
          + [pltpu.SemaphoreType.DMA] * K,
    )


_gathers = {seg_e: _make_gather(seg_e) for seg_e in set(SEG_ES)}


def _scatter_body(m0_hbm, m1_hbm, m2_hbm, r0_hbm, r1_hbm, r2_hbm, out_hbm,
                  idxr_v, mbuf_v, mbuf2_v, acc_s, sem):
    sid = lax.axis_index("s")

    def zrow(r, carry):
        for c8 in range(D // LANES):
            mbuf_v[r, pl.ds(c8 * LANES, LANES)] = jnp.zeros((LANES,), jnp.float32)
        return carry

    lax.fori_loop(0, S_CH, zrow, 0)

    def zchunk(k, carry):
        pltpu.sync_copy(mbuf_v, acc_s.at[pl.ds(sid * NPT + k * S_CH, S_CH)])
        return carry

    lax.fori_loop(0, NPT_F, zchunk, 0)
    if NPT_T:
        pltpu.sync_copy(mbuf_v.at[pl.ds(0, NPT_T)],
                        acc_s.at[pl.ds(sid * NPT + NPT_F * S_CH, NPT_T)])
    plsc.subcore_barrier()

    for m_hbm, rowi_hbm, seg_e in ((m0_hbm, r0_hbm, SEG_ES[0]),
                                   (m1_hbm, r1_hbm, SEG_ES[1]),
                                   (m2_hbm, r2_hbm, SEG_ES[2])):
        s_ept = seg_e // NS
        s_nch = s_ept // S_CH
        base = sid * s_ept
        pltpu.sync_copy(rowi_hbm.at[sid], idxr_v.at[pl.ds(0, s_nch)])
        pltpu.async_copy(m_hbm.at[pl.ds(base, S_CH)], mbuf_v, sem)

        def chunk(j, carry, m_hbm=m_hbm, base=base, s_nch=s_nch):
            nxt = jnp.minimum(j + 1, s_nch - 1)
            cur_is_0 = lax.rem(j, 2) == 0

            @pl.when(cur_is_0)
            def _():
                pltpu.make_async_copy(
                    m_hbm.at[pl.ds(base, S_CH)], mbuf_v, sem).wait()
                pltpu.async_copy(
                    m_hbm.at[pl.ds(base + nxt * S_CH, S_CH)], mbuf2_v, sem)
                pltpu.sync_copy(mbuf_v, acc_s.at[idxr_v.at[j]], add=True)

            @pl.when(jnp.logical_not(cur_is_0))
            def _():
                pltpu.make_async_copy(
                    m_hbm.at[pl.ds(base, S_CH)], mbuf2_v, sem).wait()
                pltpu.async_copy(
                    m_hbm.at[pl.ds(base + nxt * S_CH, S_CH)], mbuf_v, sem)
                pltpu.sync_copy(mbuf2_v, acc_s.at[idxr_v.at[j]], add=True)

            return carry

        lax.fori_loop(0, s_nch, chunk, 0)
        pltpu.make_async_copy(
            m_hbm.at[pl.ds(base, S_CH)],
            mbuf_v if s_nch % 2 == 0 else mbuf2_v, sem).wait()

    plsc.subcore_barrier()

    def dchunk(k, carry):
        pltpu.sync_copy(acc_s.at[pl.ds(sid * NPT + k * S_CH, S_CH)], mbuf_v)
        pltpu.sync_copy(mbuf_v, out_hbm.at[pl.ds(sid * NPT + k * S_CH, S_CH)])
        return carry

    lax.fori_loop(0, NPT_F, dchunk, 0)
    if NPT_T:
        pltpu.sync_copy(acc_s.at[pl.ds(sid * NPT + NPT_F * S_CH, NPT_T)],
                        mbuf_v.at[pl.ds(0, NPT_T)])
        pltpu.sync_copy(mbuf_v.at[pl.ds(0, NPT_T)],
                        out_hbm.at[pl.ds(sid * NPT + NPT_F * S_CH, NPT_T)])


_scatter = pl.kernel(
    _scatter_body,
    out_type=jax.ShapeDtypeStruct((N_PAD, D), jnp.float32),
    mesh=_mesh1,
    scratch_types=[
        pltpu.VMEM((S_NCH_MAX, S_CH), jnp.int32),
        pltpu.VMEM((S_CH, D), jnp.float32),
        pltpu.VMEM((S_CH, D), jnp.float32),
        pltpu.MemorySpace.VMEM_SHARED((N_PAD, D), jnp.float32),
        pltpu.SemaphoreType.DMA,
    ],
)


def _precompute_body(h_ref, w1a_ref, w1b_ref, b1_ref, a_ref, b_ref):
    hh = h_ref[...]
    a_ref[...] = jnp.dot(hh, w1a_ref[...],
                         preferred_element_type=jnp.float32) + b1_ref[...]
    b_ref[...] = jnp.dot(hh, w1b_ref[...], preferred_element_type=jnp.float32)


def _edge_mlp_body(pre_ref, ea_ref, w1c_ref, w2_ref, b2_ref, wat_ref, ba_ref,
                   out_ref):
    x = pre_ref[...] + jnp.dot(ea_ref[...], w1c_ref[...],
                               preferred_element_type=jnp.float32)
    m = jnp.maximum(x, 0.0)
    m = jnp.maximum(
        jnp.dot(m, w2_ref[...], preferred_element_type=jnp.float32)
        + b2_ref[...], 0.0)
    logit = jnp.sum(m * wat_ref[...], axis=1, keepdims=True) + ba_ref[...]
    out_ref[...] = m * jax.nn.sigmoid(logit)


def _node_body(h_ref, p_ref, wu1h_ref, wu1a_ref, bu1_ref, wu2_ref, bu2_ref,
               out_ref):
    hh = h_ref[...]
    u = jnp.maximum(
        jnp.dot(hh, wu1h_ref[...], preferred_element_type=jnp.float32)
        + jnp.dot(p_ref[...] * (1.0 / NORM), wu1a_ref[...],
                  preferred_element_type=jnp.float32)
        + bu1_ref[...], 0.0)
    out_ref[...] = hh + jnp.dot(u, wu2_ref[...],
                                preferred_element_type=jnp.float32) + bu2_ref[...]


BE = 8000
BN = 1000


def kernel(h, edge_index, edge_attr, W1, b1, W2, b2, Wa, ba, Wu1, bu1, Wu2, bu2):
    W1a, W1b, W1c = W1[:D], W1[D:2 * D], W1[2 * D:]
    Wu1h, Wu1a = Wu1[:D], Wu1[D:]
    row = edge_index[0]
    col = edge_index[1]
    seg_lo = [sum(SEG_ES[:s]) for s in range(NSEG)]
    row_g = [row[seg_lo[s]:seg_lo[s] + SEG_ES[s]].reshape(
        NC, NS, SEG_ES[s] // NW // CH, CH) for s in range(NSEG)]
    col_g = [col[seg_lo[s]:seg_lo[s] + SEG_ES[s]].reshape(
        NC, NS, SEG_ES[s] // NW // CH, CH) for s in range(NSEG)]
    row_s = [row[seg_lo[s]:seg_lo[s] + SEG_ES[s]].reshape(
        NS, SEG_ES[s] // NS // S_CH, S_CH) for s in range(NSEG)]

    A, B = pl.pallas_call(
        _precompute_body,
        out_shape=[jax.ShapeDtypeStruct((N_NODES, D), jnp.float32),
                   jax.ShapeDtypeStruct((N_NODES, D), jnp.float32)],
    )(h, W1a, W1b, b1.reshape(1, H))

    zero = lambda i: (0, 0)

    def edge_mlp_seg(pre_s, seg_e, ea_off):
        return pl.pallas_call(
            _edge_mlp_body,
            grid=(seg_e // BE,),
            in_specs=[
                pl.BlockSpec((BE, D), lambda i: (i, 0)),
                pl.BlockSpec((BE, DE), lambda i: (i + ea_off, 0)),
                pl.BlockSpec((DE, H), zero),
                pl.BlockSpec((H, H), zero),
                pl.BlockSpec((1, H), zero),
                pl.BlockSpec((1, H), zero),
                pl.BlockSpec((1, 1), zero),
            ],
            out_specs=pl.BlockSpec((BE, D), lambda i: (i, 0)),
            out_shape=jax.ShapeDtypeStruct((seg_e, D), jnp.float32),
        )(pre_s, edge_attr, W1c, W2, b2.reshape(1, H), Wa.reshape(1, H),
          ba.reshape(1, 1))

    mms = []
    for s in range(NSEG):
        pre_s = _gathers[SEG_ES[s]](A, B, row_g[s], col_g[s])
        mms.append(edge_mlp_seg(pre_s, SEG_ES[s], seg_lo[s] // BE))

    agg = _scatter(*mms, *row_s)[:N_NODES]

    out = pl.pallas_call(
        _node_body,
        grid=(N_NODES // BN,),
        in_specs=[
            pl.BlockSpec((BN, D), lambda i: (i, 0)),
            pl.BlockSpec((BN, D), lambda i: (i, 0)),
            pl.BlockSpec((H, H), zero),
            pl.BlockSpec((H, H), zero),
            pl.BlockSpec((1, H), zero),
            pl.BlockSpec((H, H), zero),
            pl.BlockSpec((1, H), zero),
        ],
        out_specs=pl.BlockSpec((BN, D), lambda i: (i, 0)),
        out_shape=jax.ShapeDtypeStruct((N_NODES, D), jnp.float32),
    )(h, agg, Wu1h, Wu1a, bu1.reshape(1, H), Wu2, bu2.reshape(1, H))

    return out
